# Initial kernel scaffold; baseline (speedup 1.0000x reference)
import jax
import jax.numpy as jnp
from jax.experimental import pallas as pl


def kernel(x, edge_index, edge_weight, W1, b1, W2, b2):
    raise NotImplementedError("write your pallas kernel here")


# speedup vs baseline: 9.2763x; 9.2763x over previous
import functools

import jax
import jax.numpy as jnp
from jax import lax
from jax.experimental import pallas as pl
from jax.experimental.pallas import tpu as pltpu
from jax.experimental.pallas import tpu_sc as plsc

N = 10000
E = 320000
D = 128

NC = 2
NS = 16
NW = NC * NS

NPAD = 10240
CHUNK = 128
NCHUNK = 79
E_W = NCHUNK * CHUNK
EPAD = NW * E_W

ROWS_PER_TILE = NPAD // NS
ROWS_PER_WORKER = NPAD // NW

_mesh = lambda: plsc.VectorSubcoreMesh(
    core_axis_name="c", subcore_axis_name="s", num_cores=NC, num_subcores=NS)
_SC_PARAMS = pltpu.CompilerParams(needs_layout_passes=False)


def _rsqrt16(x):
  i = plsc.bitcast(x, jnp.int32)
  i = 0x5F3759DF - lax.shift_right_arithmetic(i, 1)
  y = plsc.bitcast(i, jnp.float32)
  for _ in range(4):
    y = y * (1.5 - 0.5 * x * y * y)
  return y


@functools.partial(
    pl.kernel,
    out_type=jax.ShapeDtypeStruct((NPAD, D), jnp.float32),
    mesh=_mesh(),
    scratch_types=[
        pltpu.VMEM((NPAD,), jnp.float32),
        pltpu.VMEM((CHUNK,), jnp.int32),
        pltpu.VMEM((CHUNK,), jnp.float32),
        pltpu.VMEM((NS, ROWS_PER_TILE), jnp.float32),
        pltpu.VMEM((ROWS_PER_TILE,), jnp.float32),
        pltpu.VMEM((ROWS_PER_TILE, D), jnp.float32),
        pltpu.VMEM_SHARED((NS, NPAD), jnp.float32),
    ],
    compiler_params=_SC_PARAMS,
)
def _prep(dst_hbm, w_hbm, dinvb_hbm, deg_v, dst_v, w_v, slab_v, dinv_v,
          stage_v, shared):
  cid = lax.axis_index("c")
  sid = lax.axis_index("s")
  wid = cid * NS + sid

  @pl.loop(0, NPAD // 16)
  def _zero(i):
    deg_v[pl.ds(i * 16, 16)] = jnp.zeros((16,), jnp.float32)

  for half in range(2):
    row = sid * 2 + half

    @pl.loop(0, NCHUNK)
    def _acc(c):
      pltpu.sync_copy(dst_hbm.at[row, pl.ds(c * CHUNK, CHUNK)], dst_v)
      pltpu.sync_copy(w_hbm.at[row, pl.ds(c * CHUNK, CHUNK)], w_v)
      for j in range(CHUNK // 16):
        idx = dst_v[pl.ds(j * 16, 16)]
        val = w_v[pl.ds(j * 16, 16)]
        plsc.addupdate_scatter(deg_v, [idx], val)

  pltpu.sync_copy(deg_v, shared.at[sid])
  plsc.subcore_barrier()

  base = sid * ROWS_PER_TILE
  pltpu.sync_copy(shared.at[:, pl.ds(base, ROWS_PER_TILE)], slab_v)
  for v in range(ROWS_PER_TILE // 16):
    acc = slab_v[0, pl.ds(v * 16, 16)]
    for t in range(1, NS):
      acc = acc + slab_v[t, pl.ds(v * 16, 16)]
    deg16 = acc + 1.0
    dinv_v[pl.ds(v * 16, 16)] = _rsqrt16(deg16)

  writes_half = jnp.logical_or(
      jnp.logical_and(cid == 0, sid < NS // 2),
      jnp.logical_and(cid == 1, sid >= NS // 2))

  @pl.when(writes_half)
  def _write():
    @pl.loop(0, ROWS_PER_TILE)
    def _bcast(r):
      wb = plsc.load_gather(dinv_v, [jnp.zeros((16,), jnp.int32) + r])
      for j in range(D // 16):
        stage_v[r, pl.ds(j * 16, 16)] = wb

    pltpu.sync_copy(stage_v, dinvb_hbm.at[pl.ds(base, ROWS_PER_TILE), :])


@functools.partial(
    pl.kernel,
    out_type=jax.ShapeDtypeStruct((NC, NPAD, D), jnp.float32),
    mesh=_mesh(),
    scratch_types=[
        pltpu.VMEM((NCHUNK, CHUNK), jnp.int32),
        pltpu.VMEM((NCHUNK, CHUNK), jnp.int32),
        pltpu.VMEM((E_W,), jnp.float32),
        pltpu.VMEM((CHUNK, D), jnp.float32),
        pltpu.VMEM_SHARED((NPAD, D), jnp.float32),
        pltpu.SemaphoreType.DMA,
    ],
    compiler_params=_SC_PARAMS,
)
def _spmm(src_hbm, dst_hbm, w_hbm, hp_hbm, out_hbm, src_v, dst_v, w_v,
          rows_v, acc_sh, sem):
  cid = lax.axis_index("c")
  sid = lax.axis_index("s")
  wid = sid * NC + cid

  @pl.loop(0, CHUNK)
  def _zero(r):
    for j in range(D // 16):
      rows_v[r, pl.ds(j * 16, 16)] = jnp.zeros((16,), jnp.float32)

  for k in range(ROWS_PER_TILE // CHUNK):
    pltpu.sync_copy(
        rows_v, acc_sh.at[pl.ds(sid * ROWS_PER_TILE + k * CHUNK, CHUNK), :])

  pltpu.sync_copy(src_hbm.at[wid], src_v)
  pltpu.sync_copy(dst_hbm.at[wid], dst_v)
  pltpu.sync_copy(w_hbm.at[wid], w_v)
  plsc.subcore_barrier()

  @pl.loop(0, NCHUNK)
  def _edge_chunk(c):
    pltpu.async_copy(hp_hbm.at[src_v.at[c]], rows_v, sem).wait()

    @pl.loop(0, CHUNK)
    def _scale(r):
      wb = plsc.load_gather(w_v, [jnp.zeros((16,), jnp.int32) + c * CHUNK + r])
      for j in range(D // 16):
        rows_v[r, pl.ds(j * 16, 16)] = rows_v[r, pl.ds(j * 16, 16)] * wb

    pltpu.sync_copy(rows_v, acc_sh.at[dst_v.at[c]], add=True)

  plsc.subcore_barrier()
  for k in range(ROWS_PER_TILE // CHUNK):
    rows = pl.ds(sid * ROWS_PER_TILE + k * CHUNK, CHUNK)
    pltpu.sync_copy(acc_sh.at[rows, :], out_hbm.at[cid, rows, :])


BLK = 1024
_GRID = NPAD // BLK


def _tc_first_body(dinvb_ref, x_ref, w_ref, o_ref):
  o_ref[...] = dinvb_ref[...] * jnp.dot(
      x_ref[...], w_ref[...], preferred_element_type=jnp.float32)


def _tc_first(dinvb, xpad, W1):
  return pl.pallas_call(
      _tc_first_body,
      grid=(_GRID,),
      in_specs=[
          pl.BlockSpec((BLK, D), lambda i: (i, 0)),
          pl.BlockSpec((BLK, D), lambda i: (i, 0)),
          pl.BlockSpec((D, D), lambda i: (0, 0)),
      ],
      out_specs=pl.BlockSpec((BLK, D), lambda i: (i, 0)),
      out_shape=jax.ShapeDtypeStruct((NPAD, D), jnp.float32),
  )(dinvb, xpad, W1)


def _tc_mid_body(acc_ref, hp_ref, dinvb_ref, b_ref, w_ref, o_ref):
  h = dinvb_ref[...] * (acc_ref[0] + acc_ref[1] + hp_ref[...]) + b_ref[...]
  h = jnp.maximum(h, 0.0)
  o_ref[...] = dinvb_ref[...] * jnp.dot(
      h, w_ref[...], preferred_element_type=jnp.float32)


def _tc_mid(acc, hp1, dinvb, b1, W2):
  return pl.pallas_call(
      _tc_mid_body,
      grid=(_GRID,),
      in_specs=[
          pl.BlockSpec((NC, BLK, D), lambda i: (0, i, 0)),
          pl.BlockSpec((BLK, D), lambda i: (i, 0)),
          pl.BlockSpec((BLK, D), lambda i: (i, 0)),
          pl.BlockSpec((1, D), lambda i: (0, 0)),
          pl.BlockSpec((D, D), lambda i: (0, 0)),
      ],
      out_specs=pl.BlockSpec((BLK, D), lambda i: (i, 0)),
      out_shape=jax.ShapeDtypeStruct((NPAD, D), jnp.float32),
  )(acc, hp1, dinvb, b1, W2)


def _tc_final_body(acc_ref, hp_ref, dinvb_ref, b_ref, o_ref):
  o_ref[...] = (dinvb_ref[...] * (acc_ref[0] + acc_ref[1] + hp_ref[...])
                + b_ref[...])


def _tc_final(acc, hp2, dinvb, b2):
  return pl.pallas_call(
      _tc_final_body,
      grid=(_GRID,),
      in_specs=[
          pl.BlockSpec((NC, BLK, D), lambda i: (0, i, 0)),
          pl.BlockSpec((BLK, D), lambda i: (i, 0)),
          pl.BlockSpec((BLK, D), lambda i: (i, 0)),
          pl.BlockSpec((1, D), lambda i: (0, 0)),
      ],
      out_specs=pl.BlockSpec((BLK, D), lambda i: (i, 0)),
      out_shape=jax.ShapeDtypeStruct((NPAD, D), jnp.float32),
  )(acc, hp2, dinvb, b2)


def kernel(x, edge_index, edge_weight, W1, b1, W2, b2):
  src = edge_index[0].astype(jnp.int32)
  dst = edge_index[1].astype(jnp.int32)

  srcp = jnp.pad(src, (0, EPAD - E), constant_values=N)
  dstp = jnp.pad(dst, (0, EPAD - E), constant_values=NPAD - 1)
  wp = jnp.pad(edge_weight, (0, EPAD - E), constant_values=0.0)

  src3 = srcp.reshape(NW, NCHUNK, CHUNK)
  dst3 = dstp.reshape(NW, NCHUNK, CHUNK)
  dst2 = dstp.reshape(NW, E_W)
  w2 = wp.reshape(NW, E_W)

  xpad = jnp.pad(x, ((0, NPAD - N), (0, 0)))
  b1r = b1.reshape(1, D)
  b2r = b2.reshape(1, D)

  dinvb = _prep(dst2, w2)
  hp1 = _tc_first(dinvb, xpad, W1)
  acc1 = _spmm(src3, dst3, w2, hp1)
  hp2 = _tc_mid(acc1, hp1, dinvb, b1r, W2)
  acc2 = _spmm(src3, dst3, w2, hp2)
  out = _tc_final(acc2, hp2, dinvb, b2r)
  return out[:N]
